# baseline (speedup 1.0000x reference)
import jax
import jax.numpy as jnp
from jax.experimental import pallas as pl
from jax.experimental.pallas import tpu as pltpu

B, S, D = 2, 2048, 1024
T = 16
L = 10


def _tc_body(scal_ref, x_ref, out_ref, st_ref):
    t = pl.program_id(1)

    @pl.when(t == 0)
    def _compute_st():
        z = scal_ref[0] * x_ref[0]
        sig = jax.nn.sigmoid(z)
        st = jnp.round(L * (1.0 - sig)).astype(jnp.int32)
        st_ref[...] = jnp.clip(st, 0, T - 1)

    out_ref[0, 0] = (st_ref[...] == t).astype(jnp.float32)


def kernel(x, scaling):
    grid = (B, T)
    return pl.pallas_call(
        _tc_body,
        grid=grid,
        in_specs=[
            pl.BlockSpec(memory_space=pltpu.SMEM),
            pl.BlockSpec((1, S, D), lambda b, t: (b, 0, 0)),
        ],
        out_specs=pl.BlockSpec((1, 1, S, D), lambda b, t: (b, t, 0, 0)),
        out_shape=jax.ShapeDtypeStruct((B, T, S, D), jnp.float32),
        scratch_shapes=[pltpu.VMEM((S, D), jnp.int32)],
        compiler_params=pltpu.CompilerParams(
            vmem_limit_bytes=60 * 1024 * 1024,
        ),
    )(scaling.reshape(1), x)
